# baseline (speedup 1.0000x reference)
import functools

import jax
import jax.numpy as jnp
import numpy as np
from jax.experimental import pallas as pl
from jax.experimental.pallas import tpu as pltpu

_NPOINTS = [512, 128, 1]
_RADII = [0.2, 0.4, 0.8]
_NSAMPLE = [32, 64, 128]
_BN_EPS = 1e-5


def _fps_kernel(x_ref, y_ref, z_ref, cx_ref, cy_ref, cz_ref, *, npoint, n):
    x = x_ref[...]
    y = y_ref[...]
    z = z_ref[...]
    b = x.shape[0]
    iota = jax.lax.broadcasted_iota(jnp.int32, (b, n), 1)
    col = jax.lax.broadcasted_iota(jnp.int32, (b, npoint), 1)

    def body(i, carry):
        dists, far, ax, ay, az = carry
        onehot = iota == far
        cx = jnp.sum(jnp.where(onehot, x, 0.0), axis=1, keepdims=True)
        cy = jnp.sum(jnp.where(onehot, y, 0.0), axis=1, keepdims=True)
        cz = jnp.sum(jnp.where(onehot, z, 0.0), axis=1, keepdims=True)
        sel = col == i
        ax = jnp.where(sel, cx, ax)
        ay = jnp.where(sel, cy, ay)
        az = jnp.where(sel, cz, az)
        d = (x - cx) ** 2 + (y - cy) ** 2 + (z - cz) ** 2
        dists = jnp.minimum(dists, d)
        far = jnp.argmax(dists, axis=1, keepdims=True).astype(jnp.int32)
        return dists, far, ax, ay, az

    dists0 = jnp.full((b, n), 1e10, dtype=jnp.float32)
    far0 = jnp.zeros((b, 1), dtype=jnp.int32)
    acc0 = jnp.zeros((b, npoint), dtype=jnp.float32)
    _, _, ax, ay, az = jax.lax.fori_loop(
        0, npoint, body, (dists0, far0, acc0, acc0, acc0))
    cx_ref[...] = ax
    cy_ref[...] = ay
    cz_ref[...] = az


def _fps_call(x, y, z, npoint):
    b, n = x.shape
    out_shape = [jax.ShapeDtypeStruct((b, npoint), jnp.float32)] * 3
    return pl.pallas_call(
        functools.partial(_fps_kernel, npoint=npoint, n=n),
        out_shape=out_shape,
    )(x, y, z)


def _stage_kernel(coords_ref, pts_ref, nxyz_ref, *wb_refs, nsample, r2, n, s_t,
                  cp):
    out_ref = wb_refs[-1]
    wb = wb_refs[:-1]
    coords = coords_ref[0]
    nx = nxyz_ref[0]
    m = s_t * nsample

    dx = nx[:, 0:1] - coords[0:1, :]
    dy = nx[:, 1:2] - coords[1:2, :]
    dz = nx[:, 2:3] - coords[2:3, :]
    sqr = dx * dx + dy * dy + dz * dz
    mask = jnp.logical_not(sqr > r2)

    rank = mask.astype(jnp.int32)
    shift = 1
    while shift < n:
        shifted = jnp.concatenate(
            [jnp.zeros((s_t, shift), jnp.int32), rank[:, : n - shift]], axis=1)
        rank = rank + shifted
        shift *= 2

    rankm = jnp.where(mask, rank, 0)
    count = rank[:, n - 1:n]

    pts = pts_ref[0]
    w1 = wb[0][...]
    b1 = wb[1][...]
    ptsw = jnp.dot(pts, w1, preferred_element_type=jnp.float32)
    nxw = jnp.dot(nx, w1[0:3, :], preferred_element_type=jnp.float32)
    c1 = w1.shape[1]

    if s_t == 1:
        jcol = jax.lax.broadcasted_iota(jnp.int32, (nsample, 1), 0)
        sel_f = jnp.where(rankm == jcol + 1, 1.0, 0.0)
        g1 = jnp.dot(sel_f, ptsw, preferred_element_type=jnp.float32)
        first_e = jnp.broadcast_to(g1[0:1, :], (nsample, c1))
        nxw_e = jnp.broadcast_to(nxw, (nsample, c1))
        valid = jcol < count
    else:
        sel_f = jnp.concatenate(
            [jnp.where(rankm == j + 1, 1.0, 0.0) for j in range(nsample)],
            axis=0)
        g1 = jnp.dot(sel_f, ptsw, preferred_element_type=jnp.float32)
        first_e = jnp.concatenate([g1[0:s_t, :]] * nsample, axis=0)
        nxw_e = jnp.concatenate([nxw] * nsample, axis=0)
        valid = jnp.concatenate(
            [count > j for j in range(nsample)], axis=0)
    h = jnp.maximum(jnp.where(valid, g1, first_e) - nxw_e + b1, 0.0)

    for i in range(2, len(wb), 2):
        w = wb[i][...]
        bb = wb[i + 1][...]
        h = jnp.dot(h, w, preferred_element_type=jnp.float32) + bb
        h = jnp.maximum(h, 0.0)

    if s_t == 1:
        out_ref[0] = jnp.max(h, axis=0, keepdims=True)
    else:
        acc = h[0:s_t, :]
        for j in range(1, nsample):
            acc = jnp.maximum(acc, h[j * s_t:(j + 1) * s_t, :])
        out_ref[0] = acc


def _stage_call(coords3, pts, new_xyz, wbs, nsample, r2, s_t):
    b, _, n = coords3.shape
    cp = pts.shape[-1]
    s = new_xyz.shape[1]
    c_out = wbs[-2].shape[1]
    n_tiles = s // s_t
    grid = (b, n_tiles)

    in_specs = [
        pl.BlockSpec((1, 3, n), lambda bi, si: (bi, 0, 0)),
        pl.BlockSpec((1, n, cp), lambda bi, si: (bi, 0, 0)),
        pl.BlockSpec((1, s_t, 3), lambda bi, si: (bi, si, 0)),
    ]
    for warr in wbs:
        in_specs.append(
            pl.BlockSpec(warr.shape, lambda bi, si: (0,) * warr.ndim))

    return pl.pallas_call(
        functools.partial(_stage_kernel, nsample=nsample, r2=r2, n=n,
                          s_t=s_t, cp=cp),
        grid=grid,
        compiler_params=pltpu.CompilerParams(
            dimension_semantics=("parallel", "arbitrary")),
        in_specs=in_specs,
        out_specs=pl.BlockSpec((1, s_t, c_out), lambda bi, si: (bi, si, 0)),
        out_shape=jax.ShapeDtypeStruct((b, s, c_out), jnp.float32),
    )(coords3, pts, new_xyz, *wbs)


def _fc_kernel(f_ref, w1, b1, w2, b2, w3, b3, out_ref):
    h = f_ref[...]
    h = jnp.maximum(jnp.dot(h, w1[...], preferred_element_type=jnp.float32)
                    + b1[...], 0.0)
    h = jnp.maximum(jnp.dot(h, w2[...], preferred_element_type=jnp.float32)
                    + b2[...], 0.0)
    out_ref[...] = (jnp.dot(h, w3[...], preferred_element_type=jnp.float32)
                    + b3[...])


def _fc_call(feats, wbs, class_num):
    b = feats.shape[0]
    return pl.pallas_call(
        _fc_kernel,
        out_shape=jax.ShapeDtypeStruct((b, class_num), jnp.float32),
    )(feats, *wbs)


def _fold_bn(p):
    s = p["gamma"] / jnp.sqrt(jnp.float32(1.0 + _BN_EPS))
    return p["W"] * s[None, :], (p["b"] * s + p["beta"])[None, :]


def kernel(x, params):
    b = x.shape[0]
    xyz = jnp.transpose(x, (0, 2, 1))

    sa_wbs = []
    for layers in params["sa"]:
        wb = []
        for p in layers:
            w, bb = _fold_bn(p)
            wb.extend([w, bb])
        sa_wbs.append(wb)

    cx, cy, cz = _fps_call(x[:, 0, :], x[:, 1, :], x[:, 2, :], _NPOINTS[0])
    new_xyz1 = jnp.stack([cx, cy, cz], axis=-1)
    f1 = _stage_call(x, xyz, new_xyz1, sa_wbs[0],
                     _NSAMPLE[0], np.float32(_RADII[0] ** 2), 256)

    coords2 = jnp.stack([cx, cy, cz], axis=1)
    pts2 = jnp.concatenate([new_xyz1, f1], axis=-1)
    cx2, cy2, cz2 = _fps_call(cx, cy, cz, _NPOINTS[1])
    new_xyz2 = jnp.stack([cx2, cy2, cz2], axis=-1)
    f2 = _stage_call(coords2, pts2, new_xyz2, sa_wbs[1],
                     _NSAMPLE[1], np.float32(_RADII[1] ** 2), 128)

    coords3 = jnp.stack([cx2, cy2, cz2], axis=1)
    pts3 = jnp.concatenate([new_xyz2, f2], axis=-1)
    new_xyz3 = new_xyz2[:, 0:1, :]
    f3 = _stage_call(coords3, pts3, new_xyz3, sa_wbs[2],
                     _NSAMPLE[2], np.float32(_RADII[2] ** 2), 1)

    feats = f3.reshape(b, -1)
    fc_wbs = []
    for p in params["fc"]:
        w, bb = _fold_bn(p)
        fc_wbs.extend([w, bb])
    fc_wbs.extend([params["head"]["W"], params["head"]["b"][None, :]])
    return _fc_call(feats, fc_wbs, params["head"]["W"].shape[1])


# speedup vs baseline: 1.0548x; 1.0548x over previous
import functools

import jax
import jax.numpy as jnp
import numpy as np
from jax.experimental import pallas as pl

_NPOINTS = [512, 128, 1]
_RADII = [0.2, 0.4, 0.8]
_NSAMPLE = [32, 64, 128]
_BN_EPS = 1e-5


def _fps_loop(x, y, z, npoint):
    b, n = x.shape
    iota = jax.lax.broadcasted_iota(jnp.int32, (b, n), 1)
    col = jax.lax.broadcasted_iota(jnp.int32, (b, npoint), 1)

    def body(i, carry):
        dists, far, ax, ay, az = carry
        onehot = iota == far
        cx = jnp.sum(jnp.where(onehot, x, 0.0), axis=1, keepdims=True)
        cy = jnp.sum(jnp.where(onehot, y, 0.0), axis=1, keepdims=True)
        cz = jnp.sum(jnp.where(onehot, z, 0.0), axis=1, keepdims=True)
        sel = col == i
        ax = jnp.where(sel, cx, ax)
        ay = jnp.where(sel, cy, ay)
        az = jnp.where(sel, cz, az)
        d = (x - cx) ** 2 + (y - cy) ** 2 + (z - cz) ** 2
        dists = jnp.minimum(dists, d)
        far = jnp.argmax(dists, axis=1, keepdims=True).astype(jnp.int32)
        return dists, far, ax, ay, az

    dists0 = jnp.full((b, n), 1e10, dtype=jnp.float32)
    far0 = jnp.zeros((b, 1), dtype=jnp.int32)
    acc0 = jnp.zeros((b, npoint), dtype=jnp.float32)
    _, _, ax, ay, az = jax.lax.fori_loop(
        0, npoint, body, (dists0, far0, acc0, acc0, acc0))
    return ax, ay, az


def _fps2_kernel(x_ref, y_ref, z_ref, cx1_ref, cy1_ref, cz1_ref,
                 cx2_ref, cy2_ref, cz2_ref, *, np1, np2):
    ax, ay, az = _fps_loop(x_ref[...], y_ref[...], z_ref[...], np1)
    cx1_ref[...] = ax
    cy1_ref[...] = ay
    cz1_ref[...] = az
    bx, by, bz = _fps_loop(ax, ay, az, np2)
    cx2_ref[...] = bx
    cy2_ref[...] = by
    cz2_ref[...] = bz


def _fps2_call(x, y, z, np1, np2):
    b = x.shape[0]
    out_shape = ([jax.ShapeDtypeStruct((b, np1), jnp.float32)] * 3
                 + [jax.ShapeDtypeStruct((b, np2), jnp.float32)] * 3)
    return pl.pallas_call(
        functools.partial(_fps2_kernel, np1=np1, np2=np2),
        out_shape=out_shape,
    )(x, y, z)


def _stage_kernel(coords_ref, nxyz_ref, *f_wb_refs, nsample, r2, n, s_t,
                  cp, has_feats):
    out_ref = f_wb_refs[-1]
    if has_feats:
        feats_ref = f_wb_refs[0]
        wb = f_wb_refs[1:-1]
    else:
        feats_ref = None
        wb = f_wb_refs[:-1]
    coords = coords_ref[0]
    nx = nxyz_ref[0]
    m = s_t * nsample

    dx = nx[:, 0:1] - coords[0:1, :]
    dy = nx[:, 1:2] - coords[1:2, :]
    dz = nx[:, 2:3] - coords[2:3, :]
    sqr = dx * dx + dy * dy + dz * dz
    mask = jnp.logical_not(sqr > r2)

    rank = mask.astype(jnp.int32)
    shift = 1
    while shift < n:
        shifted = jnp.concatenate(
            [jnp.zeros((s_t, shift), jnp.int32), rank[:, : n - shift]], axis=1)
        rank = rank + shifted
        shift *= 2

    rankm = jnp.where(mask, rank, 0)
    count = rank[:, n - 1:n]

    w1 = wb[0][...]
    b1 = wb[1][...]
    ptsw = jax.lax.dot_general(
        coords, w1[0:3, :], (((0,), (0,)), ((), ())),
        preferred_element_type=jnp.float32)
    if has_feats:
        ptsw = ptsw + jnp.dot(feats_ref[0], w1[3:, :],
                              preferred_element_type=jnp.float32)
    nxw = jnp.dot(nx, w1[0:3, :], preferred_element_type=jnp.float32)
    c1 = w1.shape[1]

    if s_t == 1:
        jcol = jax.lax.broadcasted_iota(jnp.int32, (nsample, 1), 0)
        sel_f = jnp.where(rankm == jcol + 1, 1.0, 0.0)
        g1 = jnp.dot(sel_f, ptsw, preferred_element_type=jnp.float32)
        first_e = jnp.broadcast_to(g1[0:1, :], (nsample, c1))
        nxw_e = jnp.broadcast_to(nxw, (nsample, c1))
        valid = jcol < count
    else:
        sel_f = jnp.concatenate(
            [jnp.where(rankm == j + 1, 1.0, 0.0) for j in range(nsample)],
            axis=0)
        g1 = jnp.dot(sel_f, ptsw, preferred_element_type=jnp.float32)
        first_e = jnp.concatenate([g1[0:s_t, :]] * nsample, axis=0)
        nxw_e = jnp.concatenate([nxw] * nsample, axis=0)
        valid = jnp.concatenate(
            [count > j for j in range(nsample)], axis=0)
    h = jnp.maximum(jnp.where(valid, g1, first_e) - nxw_e + b1, 0.0)

    for i in range(2, len(wb), 2):
        w = wb[i][...]
        bb = wb[i + 1][...]
        h = jnp.dot(h, w, preferred_element_type=jnp.float32) + bb
        h = jnp.maximum(h, 0.0)

    if s_t == 1:
        out_ref[0] = jnp.max(h, axis=0, keepdims=True)
    else:
        acc = h[0:s_t, :]
        for j in range(1, nsample):
            acc = jnp.maximum(acc, h[j * s_t:(j + 1) * s_t, :])
        out_ref[0] = acc


def _stage_call(coords3, feats, new_xyz, wbs, nsample, r2, s_t):
    b, _, n = coords3.shape
    cp = 3 + (feats.shape[-1] if feats is not None else 0)
    s = new_xyz.shape[1]
    c_out = wbs[-2].shape[1]
    n_tiles = s // s_t
    grid = (b, n_tiles)

    in_specs = [
        pl.BlockSpec((1, 3, n), lambda bi, si: (bi, 0, 0)),
        pl.BlockSpec((1, s_t, 3), lambda bi, si: (bi, si, 0)),
    ]
    args = [coords3, new_xyz]
    if feats is not None:
        in_specs.append(
            pl.BlockSpec((1, n, feats.shape[-1]), lambda bi, si: (bi, 0, 0)))
        args.append(feats)
    for warr in wbs:
        in_specs.append(
            pl.BlockSpec(warr.shape, lambda bi, si: (0,) * warr.ndim))
    args.extend(wbs)

    return pl.pallas_call(
        functools.partial(_stage_kernel, nsample=nsample, r2=r2, n=n,
                          s_t=s_t, cp=cp, has_feats=feats is not None),
        grid=grid,
        in_specs=in_specs,
        out_specs=pl.BlockSpec((1, s_t, c_out), lambda bi, si: (bi, si, 0)),
        out_shape=jax.ShapeDtypeStruct((b, s, c_out), jnp.float32),
    )(*args)


def _fc_kernel(f_ref, w1, b1, w2, b2, w3, b3, out_ref):
    h = f_ref[...]
    h = jnp.maximum(jnp.dot(h, w1[...], preferred_element_type=jnp.float32)
                    + b1[...], 0.0)
    h = jnp.maximum(jnp.dot(h, w2[...], preferred_element_type=jnp.float32)
                    + b2[...], 0.0)
    out_ref[...] = (jnp.dot(h, w3[...], preferred_element_type=jnp.float32)
                    + b3[...])


def _fc_call(feats, wbs, class_num):
    b = feats.shape[0]
    return pl.pallas_call(
        _fc_kernel,
        out_shape=jax.ShapeDtypeStruct((b, class_num), jnp.float32),
    )(feats, *wbs)


def _fold_bn(p):
    s = p["gamma"] / jnp.sqrt(jnp.float32(1.0 + _BN_EPS))
    return p["W"] * s[None, :], (p["b"] * s + p["beta"])[None, :]


def kernel(x, params):
    b = x.shape[0]

    sa_wbs = []
    for layers in params["sa"]:
        wb = []
        for p in layers:
            w, bb = _fold_bn(p)
            wb.extend([w, bb])
        sa_wbs.append(wb)

    cx, cy, cz, cx2, cy2, cz2 = _fps2_call(
        x[:, 0, :], x[:, 1, :], x[:, 2, :], _NPOINTS[0], _NPOINTS[1])

    new_xyz1 = jnp.stack([cx, cy, cz], axis=-1)
    f1 = _stage_call(x, None, new_xyz1, sa_wbs[0],
                     _NSAMPLE[0], np.float32(_RADII[0] ** 2), 256)

    coords2 = jnp.stack([cx, cy, cz], axis=1)
    new_xyz2 = jnp.stack([cx2, cy2, cz2], axis=-1)
    f2 = _stage_call(coords2, f1, new_xyz2, sa_wbs[1],
                     _NSAMPLE[1], np.float32(_RADII[1] ** 2), 128)

    coords3 = jnp.stack([cx2, cy2, cz2], axis=1)
    new_xyz3 = new_xyz2[:, 0:1, :]
    f3 = _stage_call(coords3, f2, new_xyz3, sa_wbs[2],
                     _NSAMPLE[2], np.float32(_RADII[2] ** 2), 1)

    feats = f3.reshape(b, -1)
    fc_wbs = []
    for p in params["fc"]:
        w, bb = _fold_bn(p)
        fc_wbs.extend([w, bb])
    fc_wbs.extend([params["head"]["W"], params["head"]["b"][None, :]])
    return _fc_call(feats, fc_wbs, params["head"]["W"].shape[1])
